# baseline (speedup 1.0000x reference)
import functools

import jax
import jax.numpy as jnp
from jax import lax
from jax.experimental import pallas as pl
from jax.experimental.pallas import tpu as pltpu
from jax.experimental.pallas import tpu_sc as plsc

N_NODES = 10000
N_EDGES = 320000
D = 128

NC = 2
NS = 16
NW = NC * NS
N_PAD = 10240
E_PAD = 327680
E_PER_W = E_PAD // NW
CHUNK = 128
N_CHUNKS = E_PER_W // CHUNK
TOT_CHUNKS = E_PAD // CHUNK
C0_CHUNKS = TOT_CHUNKS // NS
C0_STAGE = 32
IDX_BUF = 32
ROWS_PER_TILE = N_PAD // NS


def _sc_mesh():
    return plsc.VectorSubcoreMesh(core_axis_name="c", subcore_axis_name="s")


def _sc_hist(dst2, zeros1):
    @functools.partial(
        pl.kernel,
        out_type=jax.ShapeDtypeStruct((NC, N_PAD), jnp.float32),
        mesh=_sc_mesh(),
        scratch_types=[
            pltpu.VMEM((N_CHUNKS, CHUNK), jnp.int32),
            pltpu.VMEM((CHUNK,), jnp.float32),
            pltpu.VMEM_SHARED((N_PAD,), jnp.float32),
        ],
    )
    def hist_kernel(dst_hbm, zeros_hbm, out_hbm, idx_v, ones_v, deg_sh):
        c = lax.axis_index("c")
        s = lax.axis_index("s")
        wid = c * NS + s
        pltpu.sync_copy(
            zeros_hbm.at[pl.ds(s * ROWS_PER_TILE, ROWS_PER_TILE)],
            deg_sh.at[pl.ds(s * ROWS_PER_TILE, ROWS_PER_TILE)],
        )
        pltpu.sync_copy(dst_hbm.at[pl.ds(wid * N_CHUNKS, N_CHUNKS)], idx_v)
        for i in range(CHUNK // 16):
            ones_v[pl.ds(i * 16, 16)] = jnp.ones((16,), jnp.float32)
        plsc.subcore_barrier()

        def body(j, carry):
            pltpu.sync_copy(ones_v, deg_sh.at[idx_v.at[j]], add=True)
            return carry

        lax.fori_loop(0, N_CHUNKS, body, 0)
        plsc.subcore_barrier()
        pltpu.sync_copy(
            deg_sh.at[pl.ds(s * ROWS_PER_TILE, ROWS_PER_TILE)],
            out_hbm.at[c, pl.ds(s * ROWS_PER_TILE, ROWS_PER_TILE)],
        )

    return hist_kernel(dst2, zeros1)


def _tc_scale_matmul(x_pad, W, degT):
    blk = 1024
    grid = N_PAD // blk

    def body(x_ref, w_ref, dp_ref, o_ref):
        deg = jnp.sum(dp_ref[...], axis=1, keepdims=True) + 1.0
        dis = lax.rsqrt(deg)
        h = jnp.dot(x_ref[...], w_ref[...], preferred_element_type=jnp.float32)
        o_ref[...] = dis * h

    return pl.pallas_call(
        body,
        grid=(grid,),
        in_specs=[
            pl.BlockSpec((blk, D), lambda i: (i, 0)),
            pl.BlockSpec((D, D), lambda i: (0, 0)),
            pl.BlockSpec((blk, NC), lambda i: (i, 0)),
        ],
        out_specs=pl.BlockSpec((blk, D), lambda i: (i, 0)),
        out_shape=jax.ShapeDtypeStruct((N_PAD, D), jnp.float32),
    )(x_pad, W, degT)


def _sc_agg(hp, src2, dst2):
    @functools.partial(
        pl.kernel,
        out_type=jax.ShapeDtypeStruct((N_PAD, D), jnp.float32),
        mesh=_sc_mesh(),
        scratch_types=[
            pltpu.VMEM((IDX_BUF, CHUNK), jnp.int32),
            pltpu.VMEM((IDX_BUF, CHUNK), jnp.int32),
            pltpu.VMEM((2, CHUNK, D), jnp.float32),
            pltpu.VMEM_SHARED((N_PAD, D), jnp.float32),
            pltpu.SemaphoreType.DMA,
            pltpu.SemaphoreType.DMA,
        ],
    )
    def agg_kernel(hp_hbm, src_hbm, dst_hbm, out_hbm,
                   src_v, dst_v, rows_v, agg_sh, sem0, sem1):
        c = lax.axis_index("c")
        s = lax.axis_index("s")
        sems = (sem0, sem1)

        def stage(base, n):
            pltpu.sync_copy(src_hbm.at[pl.ds(base, n)], src_v.at[pl.ds(0, n)])
            pltpu.sync_copy(dst_hbm.at[pl.ds(base, n)], dst_v.at[pl.ds(0, n)])
            pltpu.async_copy(hp_hbm.at[src_v.at[0]], rows_v.at[0], sem0)
            pltpu.async_copy(hp_hbm.at[src_v.at[1]], rows_v.at[1], sem1)

            def pair(jb, carry):
                for buf in range(2):
                    j = 2 * jb + buf
                    pltpu.make_async_copy(hp_hbm.at[src_v.at[j]],
                                          rows_v.at[buf], sems[buf]).wait()
                    pltpu.sync_copy(rows_v.at[buf], agg_sh.at[dst_v.at[j]],
                                    add=True)
                    pltpu.async_copy(hp_hbm.at[src_v.at[j + 2]],
                                     rows_v.at[buf], sems[buf])
                return carry

            lax.fori_loop(0, n // 2 - 1, pair, 0)
            for buf in range(2):
                j = n - 2 + buf
                pltpu.make_async_copy(hp_hbm.at[src_v.at[j]],
                                      rows_v.at[buf], sems[buf]).wait()
                pltpu.sync_copy(rows_v.at[buf], agg_sh.at[dst_v.at[j]],
                                add=True)

        @pl.when(c == 0)
        def _():
            pltpu.sync_copy(
                hp_hbm.at[pl.ds(s * ROWS_PER_TILE, ROWS_PER_TILE)],
                agg_sh.at[pl.ds(s * ROWS_PER_TILE, ROWS_PER_TILE)],
            )
            plsc.subcore_barrier()
            for st in range(C0_CHUNKS // C0_STAGE):
                stage(s * C0_CHUNKS + st * C0_STAGE, C0_STAGE)
            plsc.subcore_barrier()
            pltpu.sync_copy(
                agg_sh.at[pl.ds(s * ROWS_PER_TILE, ROWS_PER_TILE)],
                out_hbm.at[pl.ds(s * ROWS_PER_TILE, ROWS_PER_TILE)],
            )

    return agg_kernel(hp, src2, dst2)


def _tc_finish(agg, degT, b2):
    blk = 1024
    grid = N_PAD // blk

    def body(a_ref, dp_ref, b_ref, o_ref):
        deg = jnp.sum(dp_ref[...], axis=1, keepdims=True) + 1.0
        dis = lax.rsqrt(deg)
        o_ref[...] = jnp.maximum(dis * a_ref[...] + b_ref[...], 0.0)

    return pl.pallas_call(
        body,
        grid=(grid,),
        in_specs=[
            pl.BlockSpec((blk, D), lambda i: (i, 0)),
            pl.BlockSpec((blk, NC), lambda i: (i, 0)),
            pl.BlockSpec((1, D), lambda i: (0, 0)),
        ],
        out_specs=pl.BlockSpec((blk, D), lambda i: (i, 0)),
        out_shape=jax.ShapeDtypeStruct((N_PAD, D), jnp.float32),
    )(agg, degT, b2)


def kernel(x, edge_index, W, b):
    src = edge_index[0].astype(jnp.int32)
    dst = edge_index[1].astype(jnp.int32)
    n_dummy = E_PAD - N_EDGES
    pad_src = jnp.full((n_dummy,), N_NODES, jnp.int32)
    pad_dst = N_NODES + (jnp.arange(n_dummy, dtype=jnp.int32)
                         % (N_PAD - N_NODES))
    src2 = jnp.concatenate([src, pad_src]).reshape(TOT_CHUNKS, CHUNK)
    dst2 = jnp.concatenate([dst, pad_dst]).reshape(TOT_CHUNKS, CHUNK)
    x_pad = jnp.pad(x, ((0, N_PAD - N_NODES), (0, 0)))
    zeros1 = jnp.zeros((N_PAD,), jnp.float32)

    deg_parts = _sc_hist(dst2, zeros1)
    degT = deg_parts.T
    hp = _tc_scale_matmul(x_pad, W, degT)
    agg = _sc_agg(hp, src2, dst2)
    out = _tc_finish(agg, degT, b.reshape(1, D))
    return out[:N_NODES]


# speedup vs baseline: 2.7735x; 2.7735x over previous
import functools

import jax
import jax.numpy as jnp
from jax import lax
from jax.experimental import pallas as pl
from jax.experimental.pallas import tpu as pltpu
from jax.experimental.pallas import tpu_sc as plsc

N_NODES = 10000
N_EDGES = 320000
D = 128

NC = 2
NS = 16
NW = NC * NS
N_PAD = 10240
E_PAD = 327680
E_PER_W = E_PAD // NW
CHUNK = 128
N_CHUNKS = E_PER_W // CHUNK
TOT_CHUNKS = E_PAD // CHUNK
W_CHUNKS = TOT_CHUNKS // NW
W_STAGE = 40
IDX_BUF = 40
ROWS_PER_TILE = N_PAD // NS


def _sc_mesh():
    return plsc.VectorSubcoreMesh(core_axis_name="c", subcore_axis_name="s")


def _sc_hist(dst2, zeros1):
    @functools.partial(
        pl.kernel,
        out_type=jax.ShapeDtypeStruct((NC, N_PAD), jnp.float32),
        mesh=_sc_mesh(),
        scratch_types=[
            pltpu.VMEM((N_CHUNKS, CHUNK), jnp.int32),
            pltpu.VMEM((CHUNK,), jnp.float32),
            pltpu.VMEM_SHARED((N_PAD,), jnp.float32),
        ],
    )
    def hist_kernel(dst_hbm, zeros_hbm, out_hbm, idx_v, ones_v, deg_sh):
        c = lax.axis_index("c")
        s = lax.axis_index("s")
        wid = c * NS + s
        pltpu.sync_copy(
            zeros_hbm.at[pl.ds(s * ROWS_PER_TILE, ROWS_PER_TILE)],
            deg_sh.at[pl.ds(s * ROWS_PER_TILE, ROWS_PER_TILE)],
        )
        pltpu.sync_copy(dst_hbm.at[pl.ds(wid * N_CHUNKS, N_CHUNKS)], idx_v)
        for i in range(CHUNK // 16):
            ones_v[pl.ds(i * 16, 16)] = jnp.ones((16,), jnp.float32)
        plsc.subcore_barrier()

        def body(j, carry):
            pltpu.sync_copy(ones_v, deg_sh.at[idx_v.at[j]], add=True)
            return carry

        lax.fori_loop(0, N_CHUNKS, body, 0)
        plsc.subcore_barrier()
        pltpu.sync_copy(
            deg_sh.at[pl.ds(s * ROWS_PER_TILE, ROWS_PER_TILE)],
            out_hbm.at[c, pl.ds(s * ROWS_PER_TILE, ROWS_PER_TILE)],
        )

    return hist_kernel(dst2, zeros1)


def _tc_scale_matmul(x_pad, W, degT):
    blk = 1024
    grid = N_PAD // blk

    def body(x_ref, w_ref, dp_ref, o_ref):
        deg = jnp.sum(dp_ref[...], axis=1, keepdims=True) + 1.0
        dis = lax.rsqrt(deg)
        h = jnp.dot(x_ref[...], w_ref[...], preferred_element_type=jnp.float32)
        o_ref[...] = dis * h

    return pl.pallas_call(
        body,
        grid=(grid,),
        in_specs=[
            pl.BlockSpec((blk, D), lambda i: (i, 0)),
            pl.BlockSpec((D, D), lambda i: (0, 0)),
            pl.BlockSpec((blk, NC), lambda i: (i, 0)),
        ],
        out_specs=pl.BlockSpec((blk, D), lambda i: (i, 0)),
        out_shape=jax.ShapeDtypeStruct((N_PAD, D), jnp.float32),
    )(x_pad, W, degT)


def _sc_agg(hp, src2, dst2, zeros2):
    @functools.partial(
        pl.kernel,
        out_type=jax.ShapeDtypeStruct((NC, N_PAD, D), jnp.float32),
        mesh=_sc_mesh(),
        scratch_types=[
            pltpu.VMEM((IDX_BUF, CHUNK), jnp.int32),
            pltpu.VMEM((IDX_BUF, CHUNK), jnp.int32),
            pltpu.VMEM((2, CHUNK, D), jnp.float32),
            pltpu.VMEM_SHARED((N_PAD, D), jnp.float32),
            pltpu.SemaphoreType.DMA,
            pltpu.SemaphoreType.DMA,
        ],
    )
    def agg_kernel(hp_hbm, src_hbm, dst_hbm, zeros_hbm, out_hbm,
                   src_v, dst_v, rows_v, agg_sh, sem0, sem1):
        c = lax.axis_index("c")
        s = lax.axis_index("s")
        wid = c * NS + s
        sems = (sem0, sem1)

        def stage(base, n):
            pltpu.sync_copy(src_hbm.at[pl.ds(base, n)], src_v.at[pl.ds(0, n)])
            pltpu.sync_copy(dst_hbm.at[pl.ds(base, n)], dst_v.at[pl.ds(0, n)])
            pltpu.async_copy(hp_hbm.at[src_v.at[0]], rows_v.at[0], sem0)
            pltpu.async_copy(hp_hbm.at[src_v.at[1]], rows_v.at[1], sem1)

            def pair(jb, carry):
                for buf in range(2):
                    j = 2 * jb + buf
                    pltpu.make_async_copy(hp_hbm.at[src_v.at[j]],
                                          rows_v.at[buf], sems[buf]).wait()
                    pltpu.sync_copy(rows_v.at[buf], agg_sh.at[dst_v.at[j]],
                                    add=True)
                    pltpu.async_copy(hp_hbm.at[src_v.at[j + 2]],
                                     rows_v.at[buf], sems[buf])
                return carry

            lax.fori_loop(0, n // 2 - 1, pair, 0)
            for buf in range(2):
                j = n - 2 + buf
                pltpu.make_async_copy(hp_hbm.at[src_v.at[j]],
                                      rows_v.at[buf], sems[buf]).wait()
                pltpu.sync_copy(rows_v.at[buf], agg_sh.at[dst_v.at[j]],
                                add=True)

        @pl.when(c == 0)
        def _():
            pltpu.sync_copy(
                hp_hbm.at[pl.ds(s * ROWS_PER_TILE, ROWS_PER_TILE)],
                agg_sh.at[pl.ds(s * ROWS_PER_TILE, ROWS_PER_TILE)],
            )

        @pl.when(c == 1)
        def _():
            pltpu.sync_copy(
                zeros_hbm.at[pl.ds(s * ROWS_PER_TILE, ROWS_PER_TILE)],
                agg_sh.at[pl.ds(s * ROWS_PER_TILE, ROWS_PER_TILE)],
            )

        plsc.subcore_barrier()
        for st in range(W_CHUNKS // W_STAGE):
            stage(wid * W_CHUNKS + st * W_STAGE, W_STAGE)
        plsc.subcore_barrier()
        pltpu.sync_copy(
            agg_sh.at[pl.ds(s * ROWS_PER_TILE, ROWS_PER_TILE)],
            out_hbm.at[c, pl.ds(s * ROWS_PER_TILE, ROWS_PER_TILE)],
        )

    return agg_kernel(hp, src2, dst2, zeros2)


def _tc_finish(agg_parts, degT, b2):
    blk = 1024
    grid = N_PAD // blk

    def body(a_ref, dp_ref, b_ref, o_ref):
        deg = jnp.sum(dp_ref[...], axis=1, keepdims=True) + 1.0
        dis = lax.rsqrt(deg)
        total = a_ref[0] + a_ref[1]
        o_ref[...] = jnp.maximum(dis * total + b_ref[...], 0.0)

    return pl.pallas_call(
        body,
        grid=(grid,),
        in_specs=[
            pl.BlockSpec((NC, blk, D), lambda i: (0, i, 0)),
            pl.BlockSpec((blk, NC), lambda i: (i, 0)),
            pl.BlockSpec((1, D), lambda i: (0, 0)),
        ],
        out_specs=pl.BlockSpec((blk, D), lambda i: (i, 0)),
        out_shape=jax.ShapeDtypeStruct((N_PAD, D), jnp.float32),
    )(agg_parts, degT, b2)


def kernel(x, edge_index, W, b):
    src = edge_index[0].astype(jnp.int32)
    dst = edge_index[1].astype(jnp.int32)
    n_dummy = E_PAD - N_EDGES
    cyc = N_NODES + (jnp.arange(n_dummy, dtype=jnp.int32)
                     % (N_PAD - N_NODES))
    pad_src = cyc
    pad_dst = cyc
    src2 = jnp.concatenate([src, pad_src]).reshape(TOT_CHUNKS, CHUNK)
    dst2 = jnp.concatenate([dst, pad_dst]).reshape(TOT_CHUNKS, CHUNK)
    x_pad = jnp.pad(x, ((0, N_PAD - N_NODES), (0, 0)))
    zeros1 = jnp.zeros((N_PAD,), jnp.float32)
    zeros2 = jnp.zeros((N_PAD, D), jnp.float32)

    deg_parts = _sc_hist(dst2, zeros1)
    degT = deg_parts.T
    hp = _tc_scale_matmul(x_pad, W, degT)
    agg_parts = _sc_agg(hp, src2, dst2, zeros2)
    out = _tc_finish(agg_parts, degT, b.reshape(1, D))
    return out[:N_NODES]


# speedup vs baseline: 2.8997x; 1.0455x over previous
import functools

import jax
import jax.numpy as jnp
from jax import lax
from jax.experimental import pallas as pl
from jax.experimental.pallas import tpu as pltpu
from jax.experimental.pallas import tpu_sc as plsc

N_NODES = 10000
N_EDGES = 320000
D = 128

NC = 2
NS = 16
NW = NC * NS
N_PAD = 10240
E_PAD = 327680
CHUNK = 128
TOT_CHUNKS = E_PAD // CHUNK
REAL_CHUNKS = N_EDGES // CHUNK
PAD_CHUNKS = TOT_CHUNKS - REAL_CHUNKS
W_CHUNKS = TOT_CHUNKS // NW
W_STAGE = 40
IDX_BUF = 40
SEAM = NW - 1
SEAM_REAL = REAL_CHUNKS - SEAM * W_CHUNKS
ROWS_PER_TILE = N_PAD // NS


def _sc_mesh():
    return plsc.VectorSubcoreMesh(core_axis_name="c", subcore_axis_name="s")


def _sc_hist(dst_main, pad2):
    @functools.partial(
        pl.kernel,
        out_type=jax.ShapeDtypeStruct((NC, N_PAD), jnp.float32),
        mesh=_sc_mesh(),
        scratch_types=[
            pltpu.VMEM((88, CHUNK), jnp.int32),
            pltpu.VMEM((CHUNK,), jnp.float32),
            pltpu.VMEM((ROWS_PER_TILE,), jnp.float32),
            pltpu.VMEM_SHARED((N_PAD,), jnp.float32),
        ],
    )
    def hist_kernel(dst_hbm, pad_hbm, out_hbm, idx_v, ones_v, zrow_v, deg_sh):
        c = lax.axis_index("c")
        s = lax.axis_index("s")
        wid = c * NS + s
        def zb(i, carry):
            zrow_v[pl.ds(i * 16, 16)] = jnp.zeros((16,), jnp.float32)
            return carry

        lax.fori_loop(0, ROWS_PER_TILE // 16, zb, 0)
        pltpu.sync_copy(
            zrow_v, deg_sh.at[pl.ds(s * ROWS_PER_TILE, ROWS_PER_TILE)])
        for i in range(CHUNK // 16):
            ones_v[pl.ds(i * 16, 16)] = jnp.ones((16,), jnp.float32)

        @pl.when(wid != SEAM)
        def _():
            pltpu.sync_copy(dst_hbm.at[pl.ds(wid * W_CHUNKS, W_CHUNKS)],
                            idx_v.at[pl.ds(0, W_CHUNKS)])

        @pl.when(wid == SEAM)
        def _():
            pltpu.sync_copy(dst_hbm.at[pl.ds(SEAM * W_CHUNKS, SEAM_REAL)],
                            idx_v.at[pl.ds(0, SEAM_REAL)])
            pltpu.sync_copy(pad_hbm.at[pl.ds(0, PAD_CHUNKS)],
                            idx_v.at[pl.ds(24, PAD_CHUNKS)])

        plsc.subcore_barrier()

        def body(j, carry):
            pltpu.sync_copy(ones_v, deg_sh.at[idx_v.at[j]], add=True)
            return carry

        @pl.when(wid != SEAM)
        def _():
            lax.fori_loop(0, W_CHUNKS, body, 0)

        @pl.when(wid == SEAM)
        def _():
            lax.fori_loop(0, SEAM_REAL, body, 0)
            lax.fori_loop(24, 24 + PAD_CHUNKS, body, 0)

        plsc.subcore_barrier()
        pltpu.sync_copy(
            deg_sh.at[pl.ds(s * ROWS_PER_TILE, ROWS_PER_TILE)],
            out_hbm.at[c, pl.ds(s * ROWS_PER_TILE, ROWS_PER_TILE)],
        )

    return hist_kernel(dst_main, pad2)


def _tc_scale_matmul(x, W, degT):
    blk = 1000
    grid = N_NODES // blk

    def body(x_ref, w_ref, dp_ref, o_ref):
        deg = jnp.sum(dp_ref[...], axis=1, keepdims=True) + 1.0
        dis = lax.rsqrt(deg)
        h = jnp.dot(x_ref[...], w_ref[...], preferred_element_type=jnp.float32)
        o_ref[...] = dis * h

    return pl.pallas_call(
        body,
        grid=(grid,),
        in_specs=[
            pl.BlockSpec((blk, D), lambda i: (i, 0)),
            pl.BlockSpec((D, D), lambda i: (0, 0)),
            pl.BlockSpec((blk, NC), lambda i: (i, 0)),
        ],
        out_specs=pl.BlockSpec((blk, D), lambda i: (i, 0)),
        out_shape=jax.ShapeDtypeStruct((N_PAD, D), jnp.float32),
    )(x, W, degT)


def _sc_agg(hp, src_main, dst_main, pad2):
    @functools.partial(
        pl.kernel,
        out_type=jax.ShapeDtypeStruct((NC, N_PAD, D), jnp.float32),
        mesh=_sc_mesh(),
        scratch_types=[
            pltpu.VMEM((IDX_BUF, CHUNK), jnp.int32),
            pltpu.VMEM((IDX_BUF, CHUNK), jnp.int32),
            pltpu.VMEM((2, CHUNK, D), jnp.float32),
            pltpu.VMEM_SHARED((N_PAD, D), jnp.float32),
            pltpu.SemaphoreType.DMA,
            pltpu.SemaphoreType.DMA,
        ],
    )
    def agg_kernel(hp_hbm, src_hbm, dst_hbm, pad_hbm, out_hbm,
                   src_v, dst_v, rows_v, agg_sh, sem0, sem1):
        c = lax.axis_index("c")
        s = lax.axis_index("s")
        wid = c * NS + s
        sems = (sem0, sem1)

        def stage(sref, dref, base, n):
            pltpu.sync_copy(sref.at[pl.ds(base, n)], src_v.at[pl.ds(0, n)])
            pltpu.sync_copy(dref.at[pl.ds(base, n)], dst_v.at[pl.ds(0, n)])
            pltpu.async_copy(hp_hbm.at[src_v.at[0]], rows_v.at[0], sem0)
            pltpu.async_copy(hp_hbm.at[src_v.at[1]], rows_v.at[1], sem1)

            def pair(jb, carry):
                for buf in range(2):
                    j = 2 * jb + buf
                    pltpu.make_async_copy(hp_hbm.at[src_v.at[j]],
                                          rows_v.at[buf], sems[buf]).wait()
                    pltpu.sync_copy(rows_v.at[buf], agg_sh.at[dst_v.at[j]],
                                    add=True)
                    pltpu.async_copy(hp_hbm.at[src_v.at[j + 2]],
                                     rows_v.at[buf], sems[buf])
                return carry

            lax.fori_loop(0, n // 2 - 1, pair, 0)
            for buf in range(2):
                j = n - 2 + buf
                pltpu.make_async_copy(hp_hbm.at[src_v.at[j]],
                                      rows_v.at[buf], sems[buf]).wait()
                pltpu.sync_copy(rows_v.at[buf], agg_sh.at[dst_v.at[j]],
                                add=True)

        @pl.when(c == 0)
        def _():
            pltpu.sync_copy(
                hp_hbm.at[pl.ds(s * ROWS_PER_TILE, ROWS_PER_TILE)],
                agg_sh.at[pl.ds(s * ROWS_PER_TILE, ROWS_PER_TILE)],
            )

        @pl.when(c == 1)
        def _():
            def zb(r, carry):
                for i in range(D // 16):
                    rows_v[0, r, pl.ds(i * 16, 16)] = (
                        jnp.zeros((16,), jnp.float32))
                return carry

            lax.fori_loop(0, CHUNK, zb, 0)
            for k in range(ROWS_PER_TILE // CHUNK):
                pltpu.sync_copy(
                    rows_v.at[0],
                    agg_sh.at[pl.ds(s * ROWS_PER_TILE + k * CHUNK, CHUNK)])

        plsc.subcore_barrier()

        @pl.when(wid != SEAM)
        def _():
            for st in range(W_CHUNKS // W_STAGE):
                stage(src_hbm, dst_hbm, wid * W_CHUNKS + st * W_STAGE,
                      W_STAGE)

        @pl.when(wid == SEAM)
        def _():
            stage(src_hbm, dst_hbm, SEAM * W_CHUNKS, SEAM_REAL)
            stage(pad_hbm, pad_hbm, 0, W_STAGE)
            stage(pad_hbm, pad_hbm, W_STAGE, PAD_CHUNKS - W_STAGE)

        plsc.subcore_barrier()
        pltpu.sync_copy(
            agg_sh.at[pl.ds(s * ROWS_PER_TILE, ROWS_PER_TILE)],
            out_hbm.at[c, pl.ds(s * ROWS_PER_TILE, ROWS_PER_TILE)],
        )

    return agg_kernel(hp, src_main, dst_main, pad2)


def _tc_finish(agg_parts, degT, b2):
    blk = 1000
    grid = N_NODES // blk

    def body(a_ref, dp_ref, b_ref, o_ref):
        deg = jnp.sum(dp_ref[...], axis=1, keepdims=True) + 1.0
        dis = lax.rsqrt(deg)
        total = a_ref[0] + a_ref[1]
        o_ref[...] = jnp.maximum(dis * total + b_ref[...], 0.0)

    return pl.pallas_call(
        body,
        grid=(grid,),
        in_specs=[
            pl.BlockSpec((NC, blk, D), lambda i: (0, i, 0)),
            pl.BlockSpec((blk, NC), lambda i: (i, 0)),
            pl.BlockSpec((1, D), lambda i: (0, 0)),
        ],
        out_specs=pl.BlockSpec((blk, D), lambda i: (i, 0)),
        out_shape=jax.ShapeDtypeStruct((N_NODES, D), jnp.float32),
    )(agg_parts, degT, b2)


def kernel(x, edge_index, W, b):
    src_main = edge_index[0].astype(jnp.int32).reshape(REAL_CHUNKS, CHUNK)
    dst_main = edge_index[1].astype(jnp.int32).reshape(REAL_CHUNKS, CHUNK)
    pad2 = (N_NODES + jnp.arange(PAD_CHUNKS * CHUNK, dtype=jnp.int32)
            % (N_PAD - N_NODES)).reshape(PAD_CHUNKS, CHUNK)

    deg_parts = _sc_hist(dst_main, pad2)
    degT = deg_parts.T
    hp = _tc_scale_matmul(x, W, degT)
    agg_parts = _sc_agg(hp, src_main, dst_main, pad2)
    return _tc_finish(agg_parts, degT, b.reshape(1, D))


# speedup vs baseline: 3.1534x; 1.0875x over previous
import functools

import jax
import jax.numpy as jnp
from jax import lax
from jax.experimental import pallas as pl
from jax.experimental.pallas import tpu as pltpu
from jax.experimental.pallas import tpu_sc as plsc

N_NODES = 10000
N_EDGES = 320000
D = 128

NC = 2
NS = 16
NW = NC * NS
N_PAD = 10240
E_PAD = 327680
CHUNK = 128
TOT_CHUNKS = E_PAD // CHUNK
REAL_CHUNKS = N_EDGES // CHUNK
PAD_CHUNKS = TOT_CHUNKS - REAL_CHUNKS
W_CHUNKS = TOT_CHUNKS // NW
W_STAGE = 40
IDX_BUF = 40
SEAM = NW - 1
SEAM_REAL = REAL_CHUNKS - SEAM * W_CHUNKS
ROWS_PER_TILE = N_PAD // NS


def _sc_mesh():
    return plsc.VectorSubcoreMesh(core_axis_name="c", subcore_axis_name="s")


def _sc_hist(ei3, pad2):
    @functools.partial(
        pl.kernel,
        out_type=jax.ShapeDtypeStruct((NC, N_PAD), jnp.float32),
        mesh=_sc_mesh(),
        scratch_types=[
            pltpu.VMEM((88, CHUNK), jnp.int32),
            pltpu.VMEM((CHUNK,), jnp.float32),
            pltpu.VMEM((ROWS_PER_TILE,), jnp.float32),
            pltpu.VMEM_SHARED((N_PAD,), jnp.float32),
        ],
    )
    def hist_kernel(dst_hbm, pad_hbm, out_hbm, idx_v, ones_v, zrow_v, deg_sh):
        c = lax.axis_index("c")
        s = lax.axis_index("s")
        wid = c * NS + s
        def zb(i, carry):
            zrow_v[pl.ds(i * 16, 16)] = jnp.zeros((16,), jnp.float32)
            return carry

        lax.fori_loop(0, ROWS_PER_TILE // 16, zb, 0)
        pltpu.sync_copy(
            zrow_v, deg_sh.at[pl.ds(s * ROWS_PER_TILE, ROWS_PER_TILE)])
        for i in range(CHUNK // 16):
            ones_v[pl.ds(i * 16, 16)] = jnp.ones((16,), jnp.float32)

        @pl.when(wid != SEAM)
        def _():
            pltpu.sync_copy(dst_hbm.at[pl.ds(wid * W_CHUNKS, W_CHUNKS), 1],
                            idx_v.at[pl.ds(0, W_CHUNKS)])

        @pl.when(wid == SEAM)
        def _():
            pltpu.sync_copy(dst_hbm.at[pl.ds(SEAM * W_CHUNKS, SEAM_REAL), 1],
                            idx_v.at[pl.ds(0, SEAM_REAL)])
            pltpu.sync_copy(pad_hbm.at[pl.ds(0, PAD_CHUNKS)],
                            idx_v.at[pl.ds(24, PAD_CHUNKS)])

        plsc.subcore_barrier()

        def body(j, carry):
            pltpu.sync_copy(ones_v, deg_sh.at[idx_v.at[j]], add=True)
            return carry

        @pl.when(wid != SEAM)
        def _():
            lax.fori_loop(0, W_CHUNKS, body, 0)

        @pl.when(wid == SEAM)
        def _():
            lax.fori_loop(0, SEAM_REAL, body, 0)
            lax.fori_loop(24, 24 + PAD_CHUNKS, body, 0)

        plsc.subcore_barrier()
        pltpu.sync_copy(
            deg_sh.at[pl.ds(s * ROWS_PER_TILE, ROWS_PER_TILE)],
            out_hbm.at[c, pl.ds(s * ROWS_PER_TILE, ROWS_PER_TILE)],
        )

    return hist_kernel(ei3, pad2)


def _tc_scale_matmul(x, W, degT):
    blk = 1000
    grid = N_NODES // blk

    def body(x_ref, w_ref, dp_ref, o_ref):
        deg = jnp.sum(dp_ref[...], axis=1, keepdims=True) + 1.0
        dis = lax.rsqrt(deg)
        h = jnp.dot(x_ref[...], w_ref[...], preferred_element_type=jnp.float32)
        o_ref[...] = dis * h

    return pl.pallas_call(
        body,
        grid=(grid,),
        in_specs=[
            pl.BlockSpec((blk, D), lambda i: (i, 0)),
            pl.BlockSpec((D, D), lambda i: (0, 0)),
            pl.BlockSpec((blk, NC), lambda i: (i, 0)),
        ],
        out_specs=pl.BlockSpec((blk, D), lambda i: (i, 0)),
        out_shape=jax.ShapeDtypeStruct((N_PAD, D), jnp.float32),
    )(x, W, degT)


def _sc_agg(hp, ei3, pad2):
    @functools.partial(
        pl.kernel,
        out_type=jax.ShapeDtypeStruct((NC, N_PAD, D), jnp.float32),
        mesh=_sc_mesh(),
        scratch_types=[
            pltpu.VMEM((IDX_BUF, CHUNK), jnp.int32),
            pltpu.VMEM((IDX_BUF, CHUNK), jnp.int32),
            pltpu.VMEM((2, CHUNK, D), jnp.float32),
            pltpu.VMEM_SHARED((N_PAD, D), jnp.float32),
            pltpu.SemaphoreType.DMA,
            pltpu.SemaphoreType.DMA,
        ],
    )
    def agg_kernel(hp_hbm, ei_hbm, pad_hbm, out_hbm,
                   src_v, dst_v, rows_v, agg_sh, sem0, sem1):
        c = lax.axis_index("c")
        s = lax.axis_index("s")
        wid = c * NS + s
        sems = (sem0, sem1)

        def stage(sslice, dslice, base, n):
            pltpu.sync_copy(sslice(base, n), src_v.at[pl.ds(0, n)])
            pltpu.sync_copy(dslice(base, n), dst_v.at[pl.ds(0, n)])
            pltpu.async_copy(hp_hbm.at[src_v.at[0]], rows_v.at[0], sem0)
            pltpu.async_copy(hp_hbm.at[src_v.at[1]], rows_v.at[1], sem1)

            def pair(jb, carry):
                for buf in range(2):
                    j = 2 * jb + buf
                    pltpu.make_async_copy(hp_hbm.at[src_v.at[j]],
                                          rows_v.at[buf], sems[buf]).wait()
                    pltpu.sync_copy(rows_v.at[buf], agg_sh.at[dst_v.at[j]],
                                    add=True)
                    pltpu.async_copy(hp_hbm.at[src_v.at[j + 2]],
                                     rows_v.at[buf], sems[buf])
                return carry

            lax.fori_loop(0, n // 2 - 1, pair, 0)
            for buf in range(2):
                j = n - 2 + buf
                pltpu.make_async_copy(hp_hbm.at[src_v.at[j]],
                                      rows_v.at[buf], sems[buf]).wait()
                pltpu.sync_copy(rows_v.at[buf], agg_sh.at[dst_v.at[j]],
                                add=True)

        @pl.when(c == 0)
        def _():
            pltpu.sync_copy(
                hp_hbm.at[pl.ds(s * ROWS_PER_TILE, ROWS_PER_TILE)],
                agg_sh.at[pl.ds(s * ROWS_PER_TILE, ROWS_PER_TILE)],
            )

        @pl.when(c == 1)
        def _():
            def zb(r, carry):
                for i in range(D // 16):
                    rows_v[0, r, pl.ds(i * 16, 16)] = (
                        jnp.zeros((16,), jnp.float32))
                return carry

            lax.fori_loop(0, CHUNK, zb, 0)
            for k in range(ROWS_PER_TILE // CHUNK):
                pltpu.sync_copy(
                    rows_v.at[0],
                    agg_sh.at[pl.ds(s * ROWS_PER_TILE + k * CHUNK, CHUNK)])

        plsc.subcore_barrier()

        def ei_src(b, n):
            return ei_hbm.at[pl.ds(b, n), 0]

        def ei_dst(b, n):
            return ei_hbm.at[pl.ds(b, n), 1]

        def pad_sl(b, n):
            return pad_hbm.at[pl.ds(b, n)]

        @pl.when(wid != SEAM)
        def _():
            for st in range(W_CHUNKS // W_STAGE):
                stage(ei_src, ei_dst, wid * W_CHUNKS + st * W_STAGE,
                      W_STAGE)

        @pl.when(wid == SEAM)
        def _():
            stage(ei_src, ei_dst, SEAM * W_CHUNKS, SEAM_REAL)
            stage(pad_sl, pad_sl, 0, W_STAGE)
            stage(pad_sl, pad_sl, W_STAGE, PAD_CHUNKS - W_STAGE)

        plsc.subcore_barrier()
        pltpu.sync_copy(
            agg_sh.at[pl.ds(s * ROWS_PER_TILE, ROWS_PER_TILE)],
            out_hbm.at[c, pl.ds(s * ROWS_PER_TILE, ROWS_PER_TILE)],
        )

    return agg_kernel(hp, ei3, pad2)


def _tc_finish(agg_parts, degT, b2):
    blk = 1000
    grid = N_NODES // blk

    def body(a_ref, dp_ref, b_ref, o_ref):
        deg = jnp.sum(dp_ref[...], axis=1, keepdims=True) + 1.0
        dis = lax.rsqrt(deg)
        total = a_ref[0] + a_ref[1]
        o_ref[...] = jnp.maximum(dis * total + b_ref[...], 0.0)

    return pl.pallas_call(
        body,
        grid=(grid,),
        in_specs=[
            pl.BlockSpec((NC, blk, D), lambda i: (0, i, 0)),
            pl.BlockSpec((blk, NC), lambda i: (i, 0)),
            pl.BlockSpec((1, D), lambda i: (0, 0)),
        ],
        out_specs=pl.BlockSpec((blk, D), lambda i: (i, 0)),
        out_shape=jax.ShapeDtypeStruct((N_NODES, D), jnp.float32),
    )(agg_parts, degT, b2)


def kernel(x, edge_index, W, b):
    ei3 = jnp.transpose(
        edge_index.astype(jnp.int32).reshape(2, REAL_CHUNKS, CHUNK),
        (1, 0, 2))
    pad2 = (N_NODES + jnp.arange(PAD_CHUNKS * CHUNK, dtype=jnp.int32)
            % (N_PAD - N_NODES)).reshape(PAD_CHUNKS, CHUNK)

    deg_parts = _sc_hist(ei3, pad2)
    degT = deg_parts.T
    hp = _tc_scale_matmul(x, W, degT)
    agg_parts = _sc_agg(hp, ei3, pad2)
    return _tc_finish(agg_parts, degT, b.reshape(1, D))
